# baseline (speedup 1.0000x reference)
import functools

import jax
import jax.numpy as jnp
from jax import lax
from jax.experimental import pallas as pl
from jax.experimental.pallas import tpu as pltpu
from jax.experimental.pallas import tpu_sc as plsc

_NC = 2
_NS = 16
_NW = _NC * _NS
_CH = 80


def _tl(N):
    return ((4 * N + 16 + 127) // 128) * 128


def _silu(x):
    return x * jax.nn.sigmoid(x)


def _tl3(N):
    return ((3 * N + 16 + 127) // 128) * 128


@functools.lru_cache(maxsize=None)
def _make_gather(N, E):
    EW = E // _NW
    CH = 96
    NCH = EW // CH
    TAIL = EW - NCH * CH
    assert EW * _NW == E and TAIL % 8 == 0
    mesh = plsc.VectorSubcoreMesh(core_axis_name="c", subcore_axis_name="s")

    @functools.partial(
        pl.kernel, mesh=mesh,
        out_type=[jax.ShapeDtypeStruct((E, 128), jnp.float32),
                  jax.ShapeDtypeStruct((E, 16), jnp.float32)],
        scratch_types=[
            pltpu.VMEM((_tl3(N),), jnp.float32),
            pltpu.VMEM((EW,), jnp.int32),
            pltpu.VMEM((EW,), jnp.int32),
            pltpu.VMEM((CH, 128), jnp.float32),
            pltpu.VMEM((CH, 128), jnp.float32),
            pltpu.VMEM((CH, 128), jnp.float32),
            pltpu.VMEM((CH, 128), jnp.float32),
            pltpu.VMEM((CH, 16), jnp.float32),
            pltpu.SemaphoreType.DMA,
            pltpu.SemaphoreType.DMA,
            pltpu.SemaphoreType.DMA,
            pltpu.SemaphoreType.DMA,
        ])
    def gather_k(a_tab, b_tab, cpos, row, col, s_out, diff_out,
                 ctab, ridx, cidx, bufa0, bufa1, bufb0, bufb1, dbuf,
                 sa0, sa1, sb0, sb1):
        cid = lax.axis_index("c")
        sid = lax.axis_index("s")
        base = (cid * _NS + sid) * EW
        pltpu.sync_copy(cpos, ctab)
        pltpu.sync_copy(row.at[pl.ds(base, EW)], ridx)
        pltpu.sync_copy(col.at[pl.ds(base, EW)], cidx)
        lane = lax.iota(jnp.int32, 16)
        bufs = ((bufa0, bufb0, sa0, sb0), (bufa1, bufb1, sa1, sb1))

        def fire(j, b):
            ba, bb, sa, sb = bufs[b]
            sl = pl.ds(j * CH, CH)
            pltpu.async_copy(a_tab.at[ridx.at[sl]], ba, sa)
            pltpu.async_copy(b_tab.at[cidx.at[sl]], bb, sb)

        def process(j, b, n):
            ba, bb, sa, sb = bufs[b]
            eo = j * CH

            def edge(g, c2):
                sl = pl.ds(eo + g * 16, 16)
                rv = ridx[sl] * 3
                cv = cidx[sl] * 3
                for ln in range(16):
                    dv = (ctab[pl.ds(rv[ln], 16)]
                          - ctab[pl.ds(cv[ln], 16)])
                    dbuf[g * 16 + ln, pl.ds(0, 16)] = jnp.where(
                        lane < 3, dv, 0.0)
                return c2
            lax.fori_loop(0, n // 16, edge, 0)
            pltpu.make_async_copy(a_tab.at[ridx.at[pl.ds(0, CH)]],
                                  ba, sa).wait()
            pltpu.make_async_copy(b_tab.at[cidx.at[pl.ds(0, CH)]],
                                  bb, sb).wait()

            def addrow(i, c2):
                for k in range(8):
                    sl = pl.ds(k * 16, 16)
                    ba[i, sl] = ba[i, sl] + bb[i, sl]
                return c2
            lax.fori_loop(0, n, addrow, 0)
            be = base + eo
            if n == CH:
                pltpu.sync_copy(ba, s_out.at[pl.ds(be, CH)])
                pltpu.sync_copy(dbuf, diff_out.at[pl.ds(be, CH)])
            else:
                pltpu.sync_copy(ba.at[pl.ds(0, n)], s_out.at[pl.ds(be, n)])
                pltpu.sync_copy(dbuf.at[pl.ds(0, n)],
                                diff_out.at[pl.ds(be, n)])

        fire(0, 0)
        fire(1, 1)

        def pair(p, carry):
            for b in range(2):
                j = 2 * p + b
                process(j, b, CH)

                @pl.when(j + 2 < NCH)
                def _():
                    fire(j + 2, b)
            return carry

        assert NCH % 2 == 0
        lax.fori_loop(0, NCH // 2, pair, 0)
        if TAIL:
            to = NCH * CH
            sl = pl.ds(to, TAIL)
            pltpu.async_copy(a_tab.at[ridx.at[sl]],
                             bufa0.at[pl.ds(0, TAIL)], sa0).wait()
            pltpu.async_copy(b_tab.at[cidx.at[sl]],
                             bufb0.at[pl.ds(0, TAIL)], sb0).wait()

            def edge_t(g, c2):
                sl2 = pl.ds(to + g * 16, 16)
                rv = ridx[sl2] * 3
                cv = cidx[sl2] * 3
                for ln in range(16):
                    dv = (ctab[pl.ds(rv[ln], 16)]
                          - ctab[pl.ds(cv[ln], 16)])
                    dbuf[g * 16 + ln, pl.ds(0, 16)] = jnp.where(
                        lane < 3, dv, 0.0)
                return c2
            lax.fori_loop(0, TAIL // 16, edge_t, 0)

            def addrow_t(i, c2):
                for k in range(8):
                    sl2 = pl.ds(k * 16, 16)
                    bufa0[i, sl2] = bufa0[i, sl2] + bufb0[i, sl2]
                return c2
            lax.fori_loop(0, TAIL, addrow_t, 0)
            be = base + to
            pltpu.sync_copy(bufa0.at[pl.ds(0, TAIL)],
                            s_out.at[pl.ds(be, TAIL)])
            pltpu.sync_copy(dbuf.at[pl.ds(0, TAIL)],
                            diff_out.at[pl.ds(be, TAIL)])

    return gather_k


@functools.lru_cache(maxsize=None)
def _make_scatter(N, E):
    EW = E // _NW
    NCHUNK = EW // _CH
    ZR = 32
    NP = ((N + _NS * ZR - 1) // (_NS * ZR)) * (_NS * ZR)
    NT = NP // _NS
    assert NT % ZR == 0
    mesh = plsc.VectorSubcoreMesh(core_axis_name="c", subcore_axis_name="s")

    NP8 = NP // 8
    assert NP8 % _NS == 0

    @functools.partial(
        pl.kernel, mesh=mesh,
        out_type=[jax.ShapeDtypeStruct((2, NP, 128), jnp.float32),
                  jax.ShapeDtypeStruct((2, NP8, 128), jnp.float32)],
        scratch_types=[
            pltpu.VMEM_SHARED((NP, 128), jnp.float32),
            pltpu.VMEM_SHARED((NP8, 128), jnp.float32),
            pltpu.VMEM((_CH, 128), jnp.float32),
            pltpu.VMEM((_CH, 128), jnp.float32),
            pltpu.VMEM((_CH * 16,), jnp.float32),
            pltpu.VMEM((_CH * 16,), jnp.float32),
            pltpu.VMEM((_CH,), jnp.int32),
            pltpu.VMEM((_CH,), jnp.int32),
            pltpu.VMEM((_CH,), jnp.int32),
            pltpu.SemaphoreType.DMA,
            pltpu.SemaphoreType.DMA,
        ])
    def scatter_k(m, trf, row, p_out, t_out, macc, tracc,
                  mb0, mb1, tb0, tb1, ib0, ib1, ibd, sm0, sm1):
        cid = lax.axis_index("c")
        sid = lax.axis_index("s")
        base = (cid * _NS + sid) * EW
        zeros16 = jnp.zeros((16,), jnp.float32)
        bufs = ((mb0, tb0, ib0, sm0), (mb1, tb1, ib1, sm1))

        def zrow(i, c2):
            for k in range(8):
                mb1[i, pl.ds(k * 16, 16)] = zeros16
            return c2
        lax.fori_loop(0, _CH, zrow, 0)
        for r in range(NT // _CH):
            pltpu.sync_copy(mb1, macc.at[pl.ds(sid * NT + r * _CH, _CH)])
        NT8 = NP8 // _NS
        pltpu.sync_copy(mb1.at[pl.ds(0, NT8)], tracc.at[pl.ds(sid * NT8,
                                                              NT8)])
        plsc.subcore_barrier()

        def fire(j, b):
            mb, tb, ib, sm = bufs[b]
            be = base + j * _CH
            pltpu.async_copy(row.at[pl.ds(be, _CH)], ib, sm)
            pltpu.async_copy(m.at[pl.ds(be, _CH)], mb, sm)
            pltpu.async_copy(trf.at[pl.ds(be * 16, _CH * 16)], tb, sm)

        def step(j, b):
            mb, tb, ib, sm = bufs[b]
            pltpu.make_async_copy(row.at[pl.ds(base, _CH)], ib, sm).wait()
            pltpu.make_async_copy(m.at[pl.ds(base, _CH)], mb, sm).wait()
            pltpu.make_async_copy(
                trf.at[pl.ds(base * 16, _CH * 16)], tb, sm).wait()
            pltpu.sync_copy(mb, macc.at[ib], add=True)
            def trrow(g, c2):
                iv = ib[pl.ds(g * 16, 16)]
                ibd[pl.ds(g * 16, 16)] = iv >> 3
                off = (iv & 7) * 16
                for ln in range(16):
                    i = g * 16 + ln
                    for k in range(8):
                        mb[i, pl.ds(k * 16, 16)] = zeros16
                    mb[i, pl.ds(off[ln], 16)] = tb[pl.ds(i * 16, 16)]
                return c2
            lax.fori_loop(0, _CH // 16, trrow, 0)
            pltpu.sync_copy(mb, tracc.at[ibd], add=True)

        fire(0, 0)
        fire(1, 1)

        def pair(p, carry):
            for b in range(2):
                j = 2 * p + b
                step(j, b)

                @pl.when(j + 2 < NCHUNK)
                def _():
                    fire(j + 2, b)
            return carry

        lax.fori_loop(0, NCHUNK // 2, pair, 0)
        step(NCHUNK - 1, 0)
        plsc.subcore_barrier()
        sl = pl.ds(sid * NT, NT)
        pltpu.sync_copy(macc.at[sl], p_out.at[cid, sl])
        sl8 = pl.ds(sid * NT8, NT8)
        pltpu.sync_copy(tracc.at[sl8], t_out.at[cid, sl8])

    return scatter_k


@functools.lru_cache(maxsize=None)
def _make_scatter_t(N, E):
    EW = E // _NW
    CHT = 2000
    NCHUNK = EW // CHT
    assert NCHUNK * CHT == EW
    mesh = plsc.VectorSubcoreMesh(core_axis_name="c", subcore_axis_name="s")

    @functools.partial(
        pl.kernel, mesh=mesh,
        out_type=jax.ShapeDtypeStruct((_NW, _tl(N)), jnp.float32),
        scratch_types=[
            pltpu.VMEM((_tl(N),), jnp.float32),
            pltpu.VMEM((CHT * 16,), jnp.float32),
            pltpu.VMEM((CHT * 16,), jnp.float32),
            pltpu.VMEM((EW,), jnp.int32),
            pltpu.SemaphoreType.DMA,
            pltpu.SemaphoreType.DMA,
        ])
    def scatter_t_k(trf, row, t_out, tacc, tb0, tb1, idx, st0, st1):
        cid = lax.axis_index("c")
        sid = lax.axis_index("s")
        wid = cid * _NS + sid
        base = wid * EW
        zeros16 = jnp.zeros((16,), jnp.float32)
        bufs = ((tb0, st0), (tb1, st1))

        def fire(j, b):
            tb, st = bufs[b]
            pltpu.async_copy(
                trf.at[pl.ds((base + j * CHT) * 16, CHT * 16)], tb, st)

        fire(0, 0)
        fire(1, 1)
        pltpu.sync_copy(row.at[pl.ds(base, EW)], idx)

        def tz(g, c2):
            tacc[pl.ds(g * 16, 16)] = zeros16
            return c2
        lax.fori_loop(0, _tl(N) // 16, tz, 0)

        for j in range(NCHUNK):
            b = j % 2
            tb, st = bufs[b]
            pltpu.make_async_copy(
                trf.at[pl.ds(base * 16, CHT * 16)], tb, st).wait()

            def edge(g, c2):
                iv = idx[pl.ds(j * CHT + g * 16, 16)] * 4
                for ln in range(16):
                    r4 = iv[ln]
                    tv = tacc[pl.ds(r4, 16)]
                    tacc[pl.ds(r4, 16)] = (
                        tv + tb[pl.ds((g * 16 + ln) * 16, 16)])
                return c2
            lax.fori_loop(0, CHT // 16, edge, 0)
            if j + 2 < NCHUNK:
                fire(j + 2, b)
        pltpu.sync_copy(tacc, t_out.at[wid])

    return scatter_t_k


def _dot(a, b, dims):
    return lax.dot_general(a, b, (dims, ((), ())),
                           preferred_element_type=jnp.float32)


def _linear_pallas(x, w, b, act, blk):
    n, di = x.shape
    do = w.shape[1]
    assert n % blk == 0
    b2 = b.reshape(1, do) if b is not None else jnp.zeros((1, do), jnp.float32)

    def body(x_ref, w_ref, b_ref, o_ref):
        y = _dot(x_ref[...], w_ref[...], ((1,), (0,))) + b_ref[...]
        if act == "silu":
            y = _silu(y)
        elif act == "relu":
            y = jnp.maximum(y, 0.0)
        o_ref[...] = y

    return pl.pallas_call(
        body,
        grid=(n // blk,),
        in_specs=[pl.BlockSpec((blk, di), lambda i: (i, 0)),
                  pl.BlockSpec((di, do), lambda i: (0, 0)),
                  pl.BlockSpec((1, do), lambda i: (0, 0))],
        out_specs=pl.BlockSpec((blk, do), lambda i: (i, 0)),
        out_shape=jax.ShapeDtypeStruct((n, do), jnp.float32),
    )(x, w, b2)


def _fc_head(x, p, blk=2000):
    n, di = x.shape
    w1, b1 = p[0]["W"], p[0]["b"].reshape(1, -1)
    w2, b2 = p[1]["W"], p[1]["b"].reshape(1, -1)
    dm, do = w1.shape[1], w2.shape[1]

    def body(x_ref, w1_ref, b1_ref, w2_ref, b2_ref, o_ref):
        y = jnp.maximum(
            _dot(x_ref[...], w1_ref[...], ((1,), (0,))) + b1_ref[...], 0.0)
        o_ref[...] = _dot(y, w2_ref[...], ((1,), (0,))) + b2_ref[...]

    return pl.pallas_call(
        body,
        grid=(n // blk,),
        in_specs=[pl.BlockSpec((blk, di), lambda i: (i, 0)),
                  pl.BlockSpec((di, dm), lambda i: (0, 0)),
                  pl.BlockSpec((1, dm), lambda i: (0, 0)),
                  pl.BlockSpec((dm, do), lambda i: (0, 0)),
                  pl.BlockSpec((1, do), lambda i: (0, 0))],
        out_specs=pl.BlockSpec((blk, do), lambda i: (i, 0)),
        out_shape=jax.ShapeDtypeStruct((n, do), jnp.float32),
    )(x, w1, b1, w2, b2)


def _ab_proj(h, wa, wb, blk=2000):
    n = h.shape[0]

    def body(h_ref, wa_ref, wb_ref, oa_ref, ob_ref):
        hv = h_ref[...]
        oa_ref[...] = _dot(hv, wa_ref[...], ((1,), (0,)))
        ob_ref[...] = _dot(hv, wb_ref[...], ((1,), (0,)))

    return pl.pallas_call(
        body,
        grid=(n // blk,),
        in_specs=[pl.BlockSpec((blk, 128), lambda i: (i, 0)),
                  pl.BlockSpec((128, 128), lambda i: (0, 0)),
                  pl.BlockSpec((128, 128), lambda i: (0, 0))],
        out_specs=[pl.BlockSpec((blk, 128), lambda i: (i, 0)),
                   pl.BlockSpec((blk, 128), lambda i: (i, 0))],
        out_shape=[jax.ShapeDtypeStruct((n, 128), jnp.float32),
                   jax.ShapeDtypeStruct((n, 128), jnp.float32)],
    )(h, wa, wb)


def _edge_mlp(s, diff, ef, w1d, w2, wc1, misc, blk=2560):
    e = s.shape[0]
    assert e % blk == 0

    def body(s_ref, d_ref, ef_ref, w1d_ref, w2_ref, wc1_ref, misc_ref,
             m_ref, tr_ref):
        dm = d_ref[...]
        lane = lax.broadcasted_iota(jnp.int32, (1, 16), 1)
        rad = jnp.sum(dm * dm, axis=1, keepdims=True)
        misc = misc_ref[...]
        m1 = (s_ref[...] + rad * misc[0:1, :]
              + _dot(ef_ref[...], w1d_ref[...], ((1,), (0,)))
              + misc[1:2, :])
        m1 = _silu(m1)
        m2 = _silu(_dot(m1, w2_ref[...], ((1,), (0,))) + misc[2:3, :])
        att = jax.nn.sigmoid(
            jnp.sum(m2 * misc[3:4, :], axis=1, keepdims=True) + misc[4, 0])
        mv = m2 * att
        cm = _silu(_dot(mv, wc1_ref[...], ((1,), (0,))) + misc[5:6, :])
        ct = jnp.tanh(jnp.sum(cm * misc[6:7, :], axis=1, keepdims=True))
        m_ref[...] = mv
        tr_ref[...] = jnp.where(lane == 3, 1.0, dm * ct)

    return pl.pallas_call(
        body,
        grid=(e // blk,),
        in_specs=[pl.BlockSpec((blk, 128), lambda i: (i, 0)),
                  pl.BlockSpec((blk, 16), lambda i: (i, 0)),
                  pl.BlockSpec((blk, 32), lambda i: (i, 0)),
                  pl.BlockSpec((32, 128), lambda i: (0, 0)),
                  pl.BlockSpec((128, 128), lambda i: (0, 0)),
                  pl.BlockSpec((128, 128), lambda i: (0, 0)),
                  pl.BlockSpec((8, 128), lambda i: (0, 0))],
        out_specs=[pl.BlockSpec((blk, 128), lambda i: (i, 0)),
                   pl.BlockSpec((blk, 16), lambda i: (i, 0))],
        out_shape=[jax.ShapeDtypeStruct((e, 128), jnp.float32),
                   jax.ShapeDtypeStruct((e, 16), jnp.float32)],
    )(s, diff, ef, w1d, w2, wc1, misc)


def _node_mlp(h, p, wa, wb, w2, bb, blk=2000):
    n = h.shape[0]

    def body(h_ref, p_ref, wa_ref, wb_ref, w2_ref, b_ref, o_ref):
        hv = h_ref[...]
        magg = p_ref[0] + p_ref[1]
        x = (_dot(hv, wa_ref[...], ((1,), (0,)))
             + _dot(magg, wb_ref[...], ((1,), (0,))) + b_ref[0:1, :])
        x = _silu(x)
        o_ref[...] = _dot(x, w2_ref[...], ((1,), (0,))) + b_ref[1:2, :] + hv

    return pl.pallas_call(
        body,
        grid=(n // blk,),
        in_specs=[pl.BlockSpec((blk, 128), lambda i: (i, 0)),
                  pl.BlockSpec((2, blk, 128), lambda i: (0, i, 0)),
                  pl.BlockSpec((128, 128), lambda i: (0, 0)),
                  pl.BlockSpec((128, 128), lambda i: (0, 0)),
                  pl.BlockSpec((128, 128), lambda i: (0, 0)),
                  pl.BlockSpec((2, 128), lambda i: (0, 0))],
        out_specs=pl.BlockSpec((blk, 128), lambda i: (i, 0)),
        out_shape=jax.ShapeDtypeStruct((n, 128), jnp.float32),
    )(h, p, wa, wb, w2, bb)


def _coord_update(c, t, blk=200):
    n = c.shape[0]

    def body(c_ref, t_ref, o_ref):
        tv = t_ref[0] + t_ref[1]
        cnt = jnp.maximum(tv[:, 3:4], 1.0)
        o_ref[...] = c_ref[...] + tv[:, :3] / cnt

    return pl.pallas_call(
        body,
        grid=(n // blk,),
        in_specs=[pl.BlockSpec((blk, 3), lambda i: (i, 0)),
                  pl.BlockSpec((2, blk, 16), lambda i: (0, i, 0))],
        out_specs=pl.BlockSpec((blk, 3), lambda i: (i, 0)),
        out_shape=jax.ShapeDtypeStruct((n, 3), jnp.float32),
    )(c, t)


def _gcl(lp, h, c, row, col, ef, N, E):
    w1 = lp["edge_mlp"][0]["W"]
    misc = jnp.zeros((8, 128), jnp.float32)
    misc = misc.at[0].set(w1[256])
    misc = misc.at[1].set(lp["edge_mlp"][0]["b"])
    misc = misc.at[2].set(lp["edge_mlp"][1]["b"])
    misc = misc.at[3].set(lp["att_mlp"]["W"][:, 0])
    misc = misc.at[4, 0].set(lp["att_mlp"]["b"][0])
    misc = misc.at[5].set(lp["coord_mlp"][0]["b"])
    misc = misc.at[6].set(lp["coord_mlp"][1]["W"][:, 0])

    a, b = _ab_proj(h, w1[:128], w1[128:256])
    cflat = jnp.pad(c.reshape(3 * N), (0, _tl3(N) - 3 * N))
    s, diff = _make_gather(N, E)(a, b, cflat, row, col)
    m, tr = _edge_mlp(s, diff, ef, w1[257:289],
                      lp["edge_mlp"][1]["W"], lp["coord_mlp"][0]["W"], misc)
    p, t = _make_scatter(N, E)(m, tr.reshape(16 * E), row)
    bb = jnp.stack([lp["node_mlp"][0]["b"], lp["node_mlp"][1]["b"]])
    wn1 = lp["node_mlp"][0]["W"]
    h = _node_mlp(h, p, wn1[:128], wn1[128:256], lp["node_mlp"][1]["W"], bb)
    c = _coord_update(c, t.reshape(2, -1, 16))
    return h, c


def _egnn(p, h, c, row, col, ef, N, E):
    h = _linear_pallas(h, p["emb_in"]["W"], p["emb_in"]["b"], "none", 2000)
    for lp in p["layers"]:
        h, c = _gcl(lp, h, c, row, col, ef, N, E)
    return _linear_pallas(h, p["emb_out"]["W"], p["emb_out"]["b"],
                          "none", 2000), c


def kernel(x_res, x_pos, edge_feat, edge_index, params):
    N = x_res.shape[0]
    E = edge_feat.shape[0]
    row = edge_index[0]
    col = edge_index[1]
    c = x_pos.astype(jnp.float32)
    ef = _linear_pallas(edge_feat, params["edge_fc"]["W"],
                        params["edge_fc"]["b"], "none", 3200)
    h1, c = _egnn(params["eg1"], x_res, c, row, col, ef, N, E)
    h2, c = _egnn(params["eg2"], h1, c, row, col, ef, N, E)
    h3, c = _egnn(params["eg3"], h2, c, row, col, ef, N, E)
    h4, c = _egnn(params["eg4"], h3, c, row, col, ef, N, E)
    out1 = _fc_head(h1, params["fc1"])
    out2 = _fc_head(h2, params["fc2"])
    out3 = _fc_head(h3, params["fc3"])
    out4 = _fc_head(h4, params["fc4"])
    return (out4, out3, out2, out1, h4, h3, h2, h1)


# speedup vs baseline: 1.0075x; 1.0075x over previous
import functools

import jax
import jax.numpy as jnp
from jax import lax
from jax.experimental import pallas as pl
from jax.experimental.pallas import tpu as pltpu
from jax.experimental.pallas import tpu_sc as plsc

_NC = 2
_NS = 16
_NW = _NC * _NS
_CH = 80


def _tl(N):
    return ((4 * N + 16 + 127) // 128) * 128


def _silu(x):
    return x * jax.nn.sigmoid(x)


def _tl3(N):
    return ((3 * N + 16 + 127) // 128) * 128


@functools.lru_cache(maxsize=None)
def _make_gather(N, E):
    EW = E // _NW
    CH = 96
    NCH = EW // CH
    TAIL = EW - NCH * CH
    assert EW * _NW == E and TAIL % 8 == 0
    mesh = plsc.VectorSubcoreMesh(core_axis_name="c", subcore_axis_name="s")

    @functools.partial(
        pl.kernel, mesh=mesh,
        out_type=[jax.ShapeDtypeStruct((E, 128), jnp.float32),
                  jax.ShapeDtypeStruct((E, 16), jnp.float32)],
        scratch_types=[
            pltpu.VMEM((_tl3(N),), jnp.float32),
            pltpu.VMEM((EW,), jnp.int32),
            pltpu.VMEM((EW,), jnp.int32),
            pltpu.VMEM((CH, 128), jnp.float32),
            pltpu.VMEM((CH, 128), jnp.float32),
            pltpu.VMEM((CH, 128), jnp.float32),
            pltpu.VMEM((CH, 128), jnp.float32),
            pltpu.VMEM((CH, 16), jnp.float32),
            pltpu.VMEM((CH, 16), jnp.float32),
            pltpu.SemaphoreType.DMA,
            pltpu.SemaphoreType.DMA,
            pltpu.SemaphoreType.DMA,
            pltpu.SemaphoreType.DMA,
            pltpu.SemaphoreType.DMA,
            pltpu.SemaphoreType.DMA,
        ])
    def gather_k(a_tab, b_tab, cpos, row, col, s_out, diff_out,
                 ctab, ridx, cidx, bufa0, bufa1, bufb0, bufb1, dbuf0, dbuf1,
                 sa0, sa1, sb0, sb1, sw0, sw1):
        cid = lax.axis_index("c")
        sid = lax.axis_index("s")
        base = (cid * _NS + sid) * EW
        pltpu.sync_copy(cpos, ctab)
        pltpu.sync_copy(row.at[pl.ds(base, EW)], ridx)
        pltpu.sync_copy(col.at[pl.ds(base, EW)], cidx)
        lane = lax.iota(jnp.int32, 16)
        bufs = ((bufa0, bufb0, sa0, sb0), (bufa1, bufb1, sa1, sb1))
        dbufs = (dbuf0, dbuf1)
        sws = (sw0, sw1)

        def wdrain(b):
            pltpu.make_async_copy(bufs[b][0], s_out.at[pl.ds(base, CH)],
                                  sws[b]).wait()
            pltpu.make_async_copy(dbufs[b], diff_out.at[pl.ds(base, CH)],
                                  sws[b]).wait()

        def fire(j, b):
            ba, bb, sa, sb = bufs[b]
            sl = pl.ds(j * CH, CH)
            pltpu.async_copy(a_tab.at[ridx.at[sl]], ba, sa)
            pltpu.async_copy(b_tab.at[cidx.at[sl]], bb, sb)

        def process(j, b, n):
            ba, bb, sa, sb = bufs[b]
            dbuf = dbufs[b]
            eo = j * CH

            def edge(g, c2):
                sl = pl.ds(eo + g * 16, 16)
                rv = ridx[sl] * 3
                cv = cidx[sl] * 3
                for ln in range(16):
                    dv = (ctab[pl.ds(rv[ln], 16)]
                          - ctab[pl.ds(cv[ln], 16)])
                    dbuf[g * 16 + ln, pl.ds(0, 16)] = jnp.where(
                        lane < 3, dv, 0.0)
                return c2
            lax.fori_loop(0, n // 16, edge, 0)
            pltpu.make_async_copy(a_tab.at[ridx.at[pl.ds(0, CH)]],
                                  ba, sa).wait()
            pltpu.make_async_copy(b_tab.at[cidx.at[pl.ds(0, CH)]],
                                  bb, sb).wait()

            def addrow(i, c2):
                for k in range(8):
                    sl = pl.ds(k * 16, 16)
                    ba[i, sl] = ba[i, sl] + bb[i, sl]
                return c2
            lax.fori_loop(0, n, addrow, 0)
            be = base + eo
            pltpu.async_copy(ba, s_out.at[pl.ds(be, CH)], sws[b])
            pltpu.async_copy(dbuf, diff_out.at[pl.ds(be, CH)], sws[b])

        fire(0, 0)
        fire(1, 1)

        def pair(p, carry):
            for b in range(2):
                j = 2 * p + b
                process(j, b, CH)

                @pl.when(j + 2 < NCH)
                def _():
                    wdrain(b)
                    fire(j + 2, b)
            return carry

        assert NCH % 2 == 0
        lax.fori_loop(0, NCH // 2, pair, 0)
        wdrain(0)
        wdrain(1)
        if TAIL:
            to = NCH * CH
            sl = pl.ds(to, TAIL)
            pltpu.async_copy(a_tab.at[ridx.at[sl]],
                             bufa0.at[pl.ds(0, TAIL)], sa0).wait()
            pltpu.async_copy(b_tab.at[cidx.at[sl]],
                             bufb0.at[pl.ds(0, TAIL)], sb0).wait()

            def edge_t(g, c2):
                sl2 = pl.ds(to + g * 16, 16)
                rv = ridx[sl2] * 3
                cv = cidx[sl2] * 3
                for ln in range(16):
                    dv = (ctab[pl.ds(rv[ln], 16)]
                          - ctab[pl.ds(cv[ln], 16)])
                    dbuf0[g * 16 + ln, pl.ds(0, 16)] = jnp.where(
                        lane < 3, dv, 0.0)
                return c2
            lax.fori_loop(0, TAIL // 16, edge_t, 0)

            def addrow_t(i, c2):
                for k in range(8):
                    sl2 = pl.ds(k * 16, 16)
                    bufa0[i, sl2] = bufa0[i, sl2] + bufb0[i, sl2]
                return c2
            lax.fori_loop(0, TAIL, addrow_t, 0)
            be = base + to
            pltpu.sync_copy(bufa0.at[pl.ds(0, TAIL)],
                            s_out.at[pl.ds(be, TAIL)])
            pltpu.sync_copy(dbuf0.at[pl.ds(0, TAIL)],
                            diff_out.at[pl.ds(be, TAIL)])

    return gather_k


@functools.lru_cache(maxsize=None)
def _make_scatter(N, E):
    EW = E // _NW
    NCHUNK = EW // _CH
    ZR = 32
    NP = ((N + _NS * ZR - 1) // (_NS * ZR)) * (_NS * ZR)
    NT = NP // _NS
    assert NT % ZR == 0
    mesh = plsc.VectorSubcoreMesh(core_axis_name="c", subcore_axis_name="s")

    NP8 = NP // 8
    assert NP8 % _NS == 0

    @functools.partial(
        pl.kernel, mesh=mesh,
        out_type=[jax.ShapeDtypeStruct((2, NP, 128), jnp.float32),
                  jax.ShapeDtypeStruct((2, NP8, 128), jnp.float32)],
        scratch_types=[
            pltpu.VMEM_SHARED((NP, 128), jnp.float32),
            pltpu.VMEM_SHARED((NP8, 128), jnp.float32),
            pltpu.VMEM((_CH, 128), jnp.float32),
            pltpu.VMEM((_CH, 128), jnp.float32),
            pltpu.VMEM((_CH * 16,), jnp.float32),
            pltpu.VMEM((_CH * 16,), jnp.float32),
            pltpu.VMEM((_CH,), jnp.int32),
            pltpu.VMEM((_CH,), jnp.int32),
            pltpu.VMEM((_CH,), jnp.int32),
            pltpu.SemaphoreType.DMA,
            pltpu.SemaphoreType.DMA,
        ])
    def scatter_k(m, trf, row, p_out, t_out, macc, tracc,
                  mb0, mb1, tb0, tb1, ib0, ib1, ibd, sm0, sm1):
        cid = lax.axis_index("c")
        sid = lax.axis_index("s")
        base = (cid * _NS + sid) * EW
        zeros16 = jnp.zeros((16,), jnp.float32)
        bufs = ((mb0, tb0, ib0, sm0), (mb1, tb1, ib1, sm1))

        def zrow(i, c2):
            for k in range(8):
                mb1[i, pl.ds(k * 16, 16)] = zeros16
            return c2
        lax.fori_loop(0, _CH, zrow, 0)
        for r in range(NT // _CH):
            pltpu.sync_copy(mb1, macc.at[pl.ds(sid * NT + r * _CH, _CH)])
        NT8 = NP8 // _NS
        pltpu.sync_copy(mb1.at[pl.ds(0, NT8)], tracc.at[pl.ds(sid * NT8,
                                                              NT8)])
        plsc.subcore_barrier()

        def fire(j, b):
            mb, tb, ib, sm = bufs[b]
            be = base + j * _CH
            pltpu.async_copy(row.at[pl.ds(be, _CH)], ib, sm)
            pltpu.async_copy(m.at[pl.ds(be, _CH)], mb, sm)
            pltpu.async_copy(trf.at[pl.ds(be * 16, _CH * 16)], tb, sm)

        def step(j, b):
            mb, tb, ib, sm = bufs[b]
            pltpu.make_async_copy(row.at[pl.ds(base, _CH)], ib, sm).wait()
            pltpu.make_async_copy(m.at[pl.ds(base, _CH)], mb, sm).wait()
            pltpu.make_async_copy(
                trf.at[pl.ds(base * 16, _CH * 16)], tb, sm).wait()
            pltpu.sync_copy(mb, macc.at[ib], add=True)
            def trrow(g, c2):
                iv = ib[pl.ds(g * 16, 16)]
                ibd[pl.ds(g * 16, 16)] = iv >> 3
                off = (iv & 7) * 16
                for ln in range(16):
                    i = g * 16 + ln
                    for k in range(8):
                        mb[i, pl.ds(k * 16, 16)] = zeros16
                    mb[i, pl.ds(off[ln], 16)] = tb[pl.ds(i * 16, 16)]
                return c2
            lax.fori_loop(0, _CH // 16, trrow, 0)
            pltpu.sync_copy(mb, tracc.at[ibd], add=True)

        fire(0, 0)
        fire(1, 1)

        def pair(p, carry):
            for b in range(2):
                j = 2 * p + b
                step(j, b)

                @pl.when(j + 2 < NCHUNK)
                def _():
                    fire(j + 2, b)
            return carry

        lax.fori_loop(0, NCHUNK // 2, pair, 0)
        step(NCHUNK - 1, 0)
        plsc.subcore_barrier()
        sl = pl.ds(sid * NT, NT)
        pltpu.sync_copy(macc.at[sl], p_out.at[cid, sl])
        sl8 = pl.ds(sid * NT8, NT8)
        pltpu.sync_copy(tracc.at[sl8], t_out.at[cid, sl8])

    return scatter_k


@functools.lru_cache(maxsize=None)
def _make_scatter_t(N, E):
    EW = E // _NW
    CHT = 2000
    NCHUNK = EW // CHT
    assert NCHUNK * CHT == EW
    mesh = plsc.VectorSubcoreMesh(core_axis_name="c", subcore_axis_name="s")

    @functools.partial(
        pl.kernel, mesh=mesh,
        out_type=jax.ShapeDtypeStruct((_NW, _tl(N)), jnp.float32),
        scratch_types=[
            pltpu.VMEM((_tl(N),), jnp.float32),
            pltpu.VMEM((CHT * 16,), jnp.float32),
            pltpu.VMEM((CHT * 16,), jnp.float32),
            pltpu.VMEM((EW,), jnp.int32),
            pltpu.SemaphoreType.DMA,
            pltpu.SemaphoreType.DMA,
        ])
    def scatter_t_k(trf, row, t_out, tacc, tb0, tb1, idx, st0, st1):
        cid = lax.axis_index("c")
        sid = lax.axis_index("s")
        wid = cid * _NS + sid
        base = wid * EW
        zeros16 = jnp.zeros((16,), jnp.float32)
        bufs = ((tb0, st0), (tb1, st1))

        def fire(j, b):
            tb, st = bufs[b]
            pltpu.async_copy(
                trf.at[pl.ds((base + j * CHT) * 16, CHT * 16)], tb, st)

        fire(0, 0)
        fire(1, 1)
        pltpu.sync_copy(row.at[pl.ds(base, EW)], idx)

        def tz(g, c2):
            tacc[pl.ds(g * 16, 16)] = zeros16
            return c2
        lax.fori_loop(0, _tl(N) // 16, tz, 0)

        for j in range(NCHUNK):
            b = j % 2
            tb, st = bufs[b]
            pltpu.make_async_copy(
                trf.at[pl.ds(base * 16, CHT * 16)], tb, st).wait()

            def edge(g, c2):
                iv = idx[pl.ds(j * CHT + g * 16, 16)] * 4
                for ln in range(16):
                    r4 = iv[ln]
                    tv = tacc[pl.ds(r4, 16)]
                    tacc[pl.ds(r4, 16)] = (
                        tv + tb[pl.ds((g * 16 + ln) * 16, 16)])
                return c2
            lax.fori_loop(0, CHT // 16, edge, 0)
            if j + 2 < NCHUNK:
                fire(j + 2, b)
        pltpu.sync_copy(tacc, t_out.at[wid])

    return scatter_t_k


def _dot(a, b, dims):
    return lax.dot_general(a, b, (dims, ((), ())),
                           preferred_element_type=jnp.float32)


def _linear_pallas(x, w, b, act, blk):
    n, di = x.shape
    do = w.shape[1]
    assert n % blk == 0
    b2 = b.reshape(1, do) if b is not None else jnp.zeros((1, do), jnp.float32)

    def body(x_ref, w_ref, b_ref, o_ref):
        y = _dot(x_ref[...], w_ref[...], ((1,), (0,))) + b_ref[...]
        if act == "silu":
            y = _silu(y)
        elif act == "relu":
            y = jnp.maximum(y, 0.0)
        o_ref[...] = y

    return pl.pallas_call(
        body,
        grid=(n // blk,),
        in_specs=[pl.BlockSpec((blk, di), lambda i: (i, 0)),
                  pl.BlockSpec((di, do), lambda i: (0, 0)),
                  pl.BlockSpec((1, do), lambda i: (0, 0))],
        out_specs=pl.BlockSpec((blk, do), lambda i: (i, 0)),
        out_shape=jax.ShapeDtypeStruct((n, do), jnp.float32),
    )(x, w, b2)


def _fc_head(x, p, blk=2000):
    n, di = x.shape
    w1, b1 = p[0]["W"], p[0]["b"].reshape(1, -1)
    w2, b2 = p[1]["W"], p[1]["b"].reshape(1, -1)
    dm, do = w1.shape[1], w2.shape[1]

    def body(x_ref, w1_ref, b1_ref, w2_ref, b2_ref, o_ref):
        y = jnp.maximum(
            _dot(x_ref[...], w1_ref[...], ((1,), (0,))) + b1_ref[...], 0.0)
        o_ref[...] = _dot(y, w2_ref[...], ((1,), (0,))) + b2_ref[...]

    return pl.pallas_call(
        body,
        grid=(n // blk,),
        in_specs=[pl.BlockSpec((blk, di), lambda i: (i, 0)),
                  pl.BlockSpec((di, dm), lambda i: (0, 0)),
                  pl.BlockSpec((1, dm), lambda i: (0, 0)),
                  pl.BlockSpec((dm, do), lambda i: (0, 0)),
                  pl.BlockSpec((1, do), lambda i: (0, 0))],
        out_specs=pl.BlockSpec((blk, do), lambda i: (i, 0)),
        out_shape=jax.ShapeDtypeStruct((n, do), jnp.float32),
    )(x, w1, b1, w2, b2)


def _ab_proj(h, wa, wb, blk=2000):
    n = h.shape[0]

    def body(h_ref, wa_ref, wb_ref, oa_ref, ob_ref):
        hv = h_ref[...]
        oa_ref[...] = _dot(hv, wa_ref[...], ((1,), (0,)))
        ob_ref[...] = _dot(hv, wb_ref[...], ((1,), (0,)))

    return pl.pallas_call(
        body,
        grid=(n // blk,),
        in_specs=[pl.BlockSpec((blk, 128), lambda i: (i, 0)),
                  pl.BlockSpec((128, 128), lambda i: (0, 0)),
                  pl.BlockSpec((128, 128), lambda i: (0, 0))],
        out_specs=[pl.BlockSpec((blk, 128), lambda i: (i, 0)),
                   pl.BlockSpec((blk, 128), lambda i: (i, 0))],
        out_shape=[jax.ShapeDtypeStruct((n, 128), jnp.float32),
                   jax.ShapeDtypeStruct((n, 128), jnp.float32)],
    )(h, wa, wb)


def _edge_mlp(s, diff, ef, w1d, w2, wc1, misc, blk=2560):
    e = s.shape[0]
    assert e % blk == 0

    def body(s_ref, d_ref, ef_ref, w1d_ref, w2_ref, wc1_ref, misc_ref,
             m_ref, tr_ref):
        dm = d_ref[...]
        lane = lax.broadcasted_iota(jnp.int32, (1, 16), 1)
        rad = jnp.sum(dm * dm, axis=1, keepdims=True)
        misc = misc_ref[...]
        m1 = (s_ref[...] + rad * misc[0:1, :]
              + _dot(ef_ref[...], w1d_ref[...], ((1,), (0,)))
              + misc[1:2, :])
        m1 = _silu(m1)
        m2 = _silu(_dot(m1, w2_ref[...], ((1,), (0,))) + misc[2:3, :])
        att = jax.nn.sigmoid(
            jnp.sum(m2 * misc[3:4, :], axis=1, keepdims=True) + misc[4, 0])
        mv = m2 * att
        cm = _silu(_dot(mv, wc1_ref[...], ((1,), (0,))) + misc[5:6, :])
        ct = jnp.tanh(jnp.sum(cm * misc[6:7, :], axis=1, keepdims=True))
        m_ref[...] = mv
        tr_ref[...] = jnp.where(lane == 3, 1.0, dm * ct)

    return pl.pallas_call(
        body,
        grid=(e // blk,),
        in_specs=[pl.BlockSpec((blk, 128), lambda i: (i, 0)),
                  pl.BlockSpec((blk, 16), lambda i: (i, 0)),
                  pl.BlockSpec((blk, 32), lambda i: (i, 0)),
                  pl.BlockSpec((32, 128), lambda i: (0, 0)),
                  pl.BlockSpec((128, 128), lambda i: (0, 0)),
                  pl.BlockSpec((128, 128), lambda i: (0, 0)),
                  pl.BlockSpec((8, 128), lambda i: (0, 0))],
        out_specs=[pl.BlockSpec((blk, 128), lambda i: (i, 0)),
                   pl.BlockSpec((blk, 16), lambda i: (i, 0))],
        out_shape=[jax.ShapeDtypeStruct((e, 128), jnp.float32),
                   jax.ShapeDtypeStruct((e, 16), jnp.float32)],
    )(s, diff, ef, w1d, w2, wc1, misc)


def _node_mlp(h, p, wa, wb, w2, bb, blk=2000):
    n = h.shape[0]

    def body(h_ref, p_ref, wa_ref, wb_ref, w2_ref, b_ref, o_ref):
        hv = h_ref[...]
        magg = p_ref[0] + p_ref[1]
        x = (_dot(hv, wa_ref[...], ((1,), (0,)))
             + _dot(magg, wb_ref[...], ((1,), (0,))) + b_ref[0:1, :])
        x = _silu(x)
        o_ref[...] = _dot(x, w2_ref[...], ((1,), (0,))) + b_ref[1:2, :] + hv

    return pl.pallas_call(
        body,
        grid=(n // blk,),
        in_specs=[pl.BlockSpec((blk, 128), lambda i: (i, 0)),
                  pl.BlockSpec((2, blk, 128), lambda i: (0, i, 0)),
                  pl.BlockSpec((128, 128), lambda i: (0, 0)),
                  pl.BlockSpec((128, 128), lambda i: (0, 0)),
                  pl.BlockSpec((128, 128), lambda i: (0, 0)),
                  pl.BlockSpec((2, 128), lambda i: (0, 0))],
        out_specs=pl.BlockSpec((blk, 128), lambda i: (i, 0)),
        out_shape=jax.ShapeDtypeStruct((n, 128), jnp.float32),
    )(h, p, wa, wb, w2, bb)


def _coord_update(c, t, blk=200):
    n = c.shape[0]

    def body(c_ref, t_ref, o_ref):
        tv = t_ref[0] + t_ref[1]
        cnt = jnp.maximum(tv[:, 3:4], 1.0)
        o_ref[...] = c_ref[...] + tv[:, :3] / cnt

    return pl.pallas_call(
        body,
        grid=(n // blk,),
        in_specs=[pl.BlockSpec((blk, 3), lambda i: (i, 0)),
                  pl.BlockSpec((2, blk, 16), lambda i: (0, i, 0))],
        out_specs=pl.BlockSpec((blk, 3), lambda i: (i, 0)),
        out_shape=jax.ShapeDtypeStruct((n, 3), jnp.float32),
    )(c, t)


def _gcl(lp, h, c, row, col, ef, N, E):
    w1 = lp["edge_mlp"][0]["W"]
    misc = jnp.zeros((8, 128), jnp.float32)
    misc = misc.at[0].set(w1[256])
    misc = misc.at[1].set(lp["edge_mlp"][0]["b"])
    misc = misc.at[2].set(lp["edge_mlp"][1]["b"])
    misc = misc.at[3].set(lp["att_mlp"]["W"][:, 0])
    misc = misc.at[4, 0].set(lp["att_mlp"]["b"][0])
    misc = misc.at[5].set(lp["coord_mlp"][0]["b"])
    misc = misc.at[6].set(lp["coord_mlp"][1]["W"][:, 0])

    a, b = _ab_proj(h, w1[:128], w1[128:256])
    cflat = jnp.pad(c.reshape(3 * N), (0, _tl3(N) - 3 * N))
    s, diff = _make_gather(N, E)(a, b, cflat, row, col)
    m, tr = _edge_mlp(s, diff, ef, w1[257:289],
                      lp["edge_mlp"][1]["W"], lp["coord_mlp"][0]["W"], misc)
    p, t = _make_scatter(N, E)(m, tr.reshape(16 * E), row)
    bb = jnp.stack([lp["node_mlp"][0]["b"], lp["node_mlp"][1]["b"]])
    wn1 = lp["node_mlp"][0]["W"]
    h = _node_mlp(h, p, wn1[:128], wn1[128:256], lp["node_mlp"][1]["W"], bb)
    c = _coord_update(c, t.reshape(2, -1, 16))
    return h, c


def _egnn(p, h, c, row, col, ef, N, E):
    h = _linear_pallas(h, p["emb_in"]["W"], p["emb_in"]["b"], "none", 2000)
    for lp in p["layers"]:
        h, c = _gcl(lp, h, c, row, col, ef, N, E)
    return _linear_pallas(h, p["emb_out"]["W"], p["emb_out"]["b"],
                          "none", 2000), c


def kernel(x_res, x_pos, edge_feat, edge_index, params):
    N = x_res.shape[0]
    E = edge_feat.shape[0]
    row = edge_index[0]
    col = edge_index[1]
    c = x_pos.astype(jnp.float32)
    ef = _linear_pallas(edge_feat, params["edge_fc"]["W"],
                        params["edge_fc"]["b"], "none", 3200)
    h1, c = _egnn(params["eg1"], x_res, c, row, col, ef, N, E)
    h2, c = _egnn(params["eg2"], h1, c, row, col, ef, N, E)
    h3, c = _egnn(params["eg3"], h2, c, row, col, ef, N, E)
    h4, c = _egnn(params["eg4"], h3, c, row, col, ef, N, E)
    out1 = _fc_head(h1, params["fc1"])
    out2 = _fc_head(h2, params["fc2"])
    out3 = _fc_head(h3, params["fc3"])
    out4 = _fc_head(h4, params["fc4"])
    return (out4, out3, out2, out1, h4, h3, h2, h1)


# speedup vs baseline: 1.0076x; 1.0002x over previous
import functools

import jax
import jax.numpy as jnp
from jax import lax
from jax.experimental import pallas as pl
from jax.experimental.pallas import tpu as pltpu
from jax.experimental.pallas import tpu_sc as plsc

_NC = 2
_NS = 16
_NW = _NC * _NS
_CH = 80


def _silu(x):
    return x * jax.nn.sigmoid(x)


def _tl3(N):
    return ((3 * N + 16 + 127) // 128) * 128


@functools.lru_cache(maxsize=None)
def _make_gather(N, E):
    EW = E // _NW
    CH = 96
    NCH = EW // CH
    TAIL = EW - NCH * CH
    assert EW * _NW == E and TAIL % 8 == 0
    mesh = plsc.VectorSubcoreMesh(core_axis_name="c", subcore_axis_name="s")

    @functools.partial(
        pl.kernel, mesh=mesh,
        out_type=[jax.ShapeDtypeStruct((E, 128), jnp.float32),
                  jax.ShapeDtypeStruct((E, 16), jnp.float32)],
        scratch_types=[
            pltpu.VMEM((_tl3(N),), jnp.float32),
            pltpu.VMEM((EW,), jnp.int32),
            pltpu.VMEM((EW,), jnp.int32),
            pltpu.VMEM((CH, 128), jnp.float32),
            pltpu.VMEM((CH, 128), jnp.float32),
            pltpu.VMEM((CH, 128), jnp.float32),
            pltpu.VMEM((CH, 128), jnp.float32),
            pltpu.VMEM((CH, 16), jnp.float32),
            pltpu.VMEM((CH, 16), jnp.float32),
            pltpu.SemaphoreType.DMA,
            pltpu.SemaphoreType.DMA,
            pltpu.SemaphoreType.DMA,
            pltpu.SemaphoreType.DMA,
            pltpu.SemaphoreType.DMA,
            pltpu.SemaphoreType.DMA,
        ])
    def gather_k(a_tab, b_tab, cpos, row, col, s_out, diff_out,
                 ctab, ridx, cidx, bufa0, bufa1, bufb0, bufb1, dbuf0, dbuf1,
                 sa0, sa1, sb0, sb1, sw0, sw1):
        cid = lax.axis_index("c")
        sid = lax.axis_index("s")
        base = (cid * _NS + sid) * EW
        pltpu.sync_copy(cpos, ctab)
        pltpu.sync_copy(row.at[pl.ds(base, EW)], ridx)
        pltpu.sync_copy(col.at[pl.ds(base, EW)], cidx)
        lane = lax.iota(jnp.int32, 16)
        bufs = ((bufa0, bufb0, sa0, sb0), (bufa1, bufb1, sa1, sb1))
        dbufs = (dbuf0, dbuf1)
        sws = (sw0, sw1)

        def wdrain(b):
            pltpu.make_async_copy(bufs[b][0], s_out.at[pl.ds(base, CH)],
                                  sws[b]).wait()
            pltpu.make_async_copy(dbufs[b], diff_out.at[pl.ds(base, CH)],
                                  sws[b]).wait()

        def fire(j, b):
            ba, bb, sa, sb = bufs[b]
            sl = pl.ds(j * CH, CH)
            pltpu.async_copy(a_tab.at[ridx.at[sl]], ba, sa)
            pltpu.async_copy(b_tab.at[cidx.at[sl]], bb, sb)

        def process(j, b, n):
            ba, bb, sa, sb = bufs[b]
            dbuf = dbufs[b]
            eo = j * CH

            def edge(g, c2):
                sl = pl.ds(eo + g * 16, 16)
                rv = ridx[sl] * 3
                cv = cidx[sl] * 3
                for ln in range(16):
                    dv = (ctab[pl.ds(rv[ln], 16)]
                          - ctab[pl.ds(cv[ln], 16)])
                    dbuf[g * 16 + ln, pl.ds(0, 16)] = jnp.where(
                        lane < 3, dv, 0.0)
                return c2
            lax.fori_loop(0, n // 16, edge, 0)
            pltpu.make_async_copy(a_tab.at[ridx.at[pl.ds(0, CH)]],
                                  ba, sa).wait()
            pltpu.make_async_copy(b_tab.at[cidx.at[pl.ds(0, CH)]],
                                  bb, sb).wait()

            def addrow(i, c2):
                for k in range(8):
                    sl = pl.ds(k * 16, 16)
                    ba[i, sl] = ba[i, sl] + bb[i, sl]
                return c2
            lax.fori_loop(0, n, addrow, 0)
            be = base + eo
            pltpu.async_copy(ba, s_out.at[pl.ds(be, CH)], sws[b])
            pltpu.async_copy(dbuf, diff_out.at[pl.ds(be, CH)], sws[b])

        fire(0, 0)
        fire(1, 1)

        def pair(p, carry):
            for b in range(2):
                j = 2 * p + b
                process(j, b, CH)

                @pl.when(j + 2 < NCH)
                def _():
                    wdrain(b)
                    fire(j + 2, b)
            return carry

        assert NCH % 2 == 0
        lax.fori_loop(0, NCH // 2, pair, 0)
        wdrain(0)
        wdrain(1)
        if TAIL:
            to = NCH * CH
            sl = pl.ds(to, TAIL)
            pltpu.async_copy(a_tab.at[ridx.at[sl]],
                             bufa0.at[pl.ds(0, TAIL)], sa0).wait()
            pltpu.async_copy(b_tab.at[cidx.at[sl]],
                             bufb0.at[pl.ds(0, TAIL)], sb0).wait()

            def edge_t(g, c2):
                sl2 = pl.ds(to + g * 16, 16)
                rv = ridx[sl2] * 3
                cv = cidx[sl2] * 3
                for ln in range(16):
                    dv = (ctab[pl.ds(rv[ln], 16)]
                          - ctab[pl.ds(cv[ln], 16)])
                    dbuf0[g * 16 + ln, pl.ds(0, 16)] = jnp.where(
                        lane < 3, dv, 0.0)
                return c2
            lax.fori_loop(0, TAIL // 16, edge_t, 0)

            def addrow_t(i, c2):
                for k in range(8):
                    sl2 = pl.ds(k * 16, 16)
                    bufa0[i, sl2] = bufa0[i, sl2] + bufb0[i, sl2]
                return c2
            lax.fori_loop(0, TAIL, addrow_t, 0)
            be = base + to
            pltpu.sync_copy(bufa0.at[pl.ds(0, TAIL)],
                            s_out.at[pl.ds(be, TAIL)])
            pltpu.sync_copy(dbuf0.at[pl.ds(0, TAIL)],
                            diff_out.at[pl.ds(be, TAIL)])

    return gather_k


@functools.lru_cache(maxsize=None)
def _make_scatter(N, E):
    EW = E // _NW
    NCHUNK = EW // _CH
    ZR = 32
    NP = ((N + _NS * ZR - 1) // (_NS * ZR)) * (_NS * ZR)
    NT = NP // _NS
    assert NT % ZR == 0
    mesh = plsc.VectorSubcoreMesh(core_axis_name="c", subcore_axis_name="s")

    NP8 = NP // 8
    assert NP8 % _NS == 0

    @functools.partial(
        pl.kernel, mesh=mesh,
        out_type=[jax.ShapeDtypeStruct((2, NP, 128), jnp.float32),
                  jax.ShapeDtypeStruct((2, NP8, 128), jnp.float32)],
        scratch_types=[
            pltpu.VMEM_SHARED((NP, 128), jnp.float32),
            pltpu.VMEM_SHARED((NP8, 128), jnp.float32),
            pltpu.VMEM((_CH, 128), jnp.float32),
            pltpu.VMEM((_CH, 128), jnp.float32),
            pltpu.VMEM((_CH * 16,), jnp.float32),
            pltpu.VMEM((_CH * 16,), jnp.float32),
            pltpu.VMEM((_CH,), jnp.int32),
            pltpu.VMEM((_CH,), jnp.int32),
            pltpu.VMEM((_CH,), jnp.int32),
            pltpu.SemaphoreType.DMA,
            pltpu.SemaphoreType.DMA,
        ])
    def scatter_k(m, trf, row, p_out, t_out, macc, tracc,
                  mb0, mb1, tb0, tb1, ib0, ib1, ibd, sm0, sm1):
        cid = lax.axis_index("c")
        sid = lax.axis_index("s")
        base = (cid * _NS + sid) * EW
        zeros16 = jnp.zeros((16,), jnp.float32)
        bufs = ((mb0, tb0, ib0, sm0), (mb1, tb1, ib1, sm1))

        def zrow(i, c2):
            for k in range(8):
                mb1[i, pl.ds(k * 16, 16)] = zeros16
            return c2
        lax.fori_loop(0, _CH, zrow, 0)
        for r in range(NT // _CH):
            pltpu.sync_copy(mb1, macc.at[pl.ds(sid * NT + r * _CH, _CH)])
        NT8 = NP8 // _NS
        pltpu.sync_copy(mb1.at[pl.ds(0, NT8)], tracc.at[pl.ds(sid * NT8,
                                                              NT8)])
        plsc.subcore_barrier()

        def fire(j, b):
            mb, tb, ib, sm = bufs[b]
            be = base + j * _CH
            pltpu.async_copy(row.at[pl.ds(be, _CH)], ib, sm)
            pltpu.async_copy(m.at[pl.ds(be, _CH)], mb, sm)
            pltpu.async_copy(trf.at[pl.ds(be * 16, _CH * 16)], tb, sm)

        def step(j, b):
            mb, tb, ib, sm = bufs[b]
            pltpu.make_async_copy(row.at[pl.ds(base, _CH)], ib, sm).wait()
            pltpu.make_async_copy(m.at[pl.ds(base, _CH)], mb, sm).wait()
            pltpu.make_async_copy(
                trf.at[pl.ds(base * 16, _CH * 16)], tb, sm).wait()
            pltpu.sync_copy(mb, macc.at[ib], add=True)
            def trrow(g, c2):
                iv = ib[pl.ds(g * 16, 16)]
                ibd[pl.ds(g * 16, 16)] = iv >> 3
                off = (iv & 7) * 16
                for ln in range(16):
                    i = g * 16 + ln
                    for k in range(8):
                        mb[i, pl.ds(k * 16, 16)] = zeros16
                    mb[i, pl.ds(off[ln], 16)] = tb[pl.ds(i * 16, 16)]
                return c2
            lax.fori_loop(0, _CH // 16, trrow, 0)
            pltpu.sync_copy(mb, tracc.at[ibd], add=True)

        fire(0, 0)
        fire(1, 1)

        def pair(p, carry):
            for b in range(2):
                j = 2 * p + b
                step(j, b)

                @pl.when(j + 2 < NCHUNK)
                def _():
                    fire(j + 2, b)
            return carry

        lax.fori_loop(0, NCHUNK // 2, pair, 0)
        step(NCHUNK - 1, 0)
        plsc.subcore_barrier()
        sl = pl.ds(sid * NT, NT)
        pltpu.sync_copy(macc.at[sl], p_out.at[cid, sl])
        sl8 = pl.ds(sid * NT8, NT8)
        pltpu.sync_copy(tracc.at[sl8], t_out.at[cid, sl8])

    return scatter_k


def _dot(a, b, dims):
    return lax.dot_general(a, b, (dims, ((), ())),
                           preferred_element_type=jnp.float32)


def _linear_pallas(x, w, b, act, blk):
    n, di = x.shape
    do = w.shape[1]
    assert n % blk == 0
    b2 = b.reshape(1, do) if b is not None else jnp.zeros((1, do), jnp.float32)

    def body(x_ref, w_ref, b_ref, o_ref):
        y = _dot(x_ref[...], w_ref[...], ((1,), (0,))) + b_ref[...]
        if act == "silu":
            y = _silu(y)
        elif act == "relu":
            y = jnp.maximum(y, 0.0)
        o_ref[...] = y

    return pl.pallas_call(
        body,
        grid=(n // blk,),
        in_specs=[pl.BlockSpec((blk, di), lambda i: (i, 0)),
                  pl.BlockSpec((di, do), lambda i: (0, 0)),
                  pl.BlockSpec((1, do), lambda i: (0, 0))],
        out_specs=pl.BlockSpec((blk, do), lambda i: (i, 0)),
        out_shape=jax.ShapeDtypeStruct((n, do), jnp.float32),
    )(x, w, b2)


def _fc_head(x, p, blk=2000):
    n, di = x.shape
    w1, b1 = p[0]["W"], p[0]["b"].reshape(1, -1)
    w2, b2 = p[1]["W"], p[1]["b"].reshape(1, -1)
    dm, do = w1.shape[1], w2.shape[1]

    def body(x_ref, w1_ref, b1_ref, w2_ref, b2_ref, o_ref):
        y = jnp.maximum(
            _dot(x_ref[...], w1_ref[...], ((1,), (0,))) + b1_ref[...], 0.0)
        o_ref[...] = _dot(y, w2_ref[...], ((1,), (0,))) + b2_ref[...]

    return pl.pallas_call(
        body,
        grid=(n // blk,),
        in_specs=[pl.BlockSpec((blk, di), lambda i: (i, 0)),
                  pl.BlockSpec((di, dm), lambda i: (0, 0)),
                  pl.BlockSpec((1, dm), lambda i: (0, 0)),
                  pl.BlockSpec((dm, do), lambda i: (0, 0)),
                  pl.BlockSpec((1, do), lambda i: (0, 0))],
        out_specs=pl.BlockSpec((blk, do), lambda i: (i, 0)),
        out_shape=jax.ShapeDtypeStruct((n, do), jnp.float32),
    )(x, w1, b1, w2, b2)


def _ab_proj(h, wa, wb, blk=2000):
    n = h.shape[0]

    def body(h_ref, wa_ref, wb_ref, oa_ref, ob_ref):
        hv = h_ref[...]
        oa_ref[...] = _dot(hv, wa_ref[...], ((1,), (0,)))
        ob_ref[...] = _dot(hv, wb_ref[...], ((1,), (0,)))

    return pl.pallas_call(
        body,
        grid=(n // blk,),
        in_specs=[pl.BlockSpec((blk, 128), lambda i: (i, 0)),
                  pl.BlockSpec((128, 128), lambda i: (0, 0)),
                  pl.BlockSpec((128, 128), lambda i: (0, 0))],
        out_specs=[pl.BlockSpec((blk, 128), lambda i: (i, 0)),
                   pl.BlockSpec((blk, 128), lambda i: (i, 0))],
        out_shape=[jax.ShapeDtypeStruct((n, 128), jnp.float32),
                   jax.ShapeDtypeStruct((n, 128), jnp.float32)],
    )(h, wa, wb)


def _edge_mlp(s, diff, ef, w1d, w2, wc1, misc, blk=2560):
    e = s.shape[0]
    assert e % blk == 0

    def body(s_ref, d_ref, ef_ref, w1d_ref, w2_ref, wc1_ref, misc_ref,
             m_ref, tr_ref):
        dm = d_ref[...]
        lane = lax.broadcasted_iota(jnp.int32, (1, 16), 1)
        rad = jnp.sum(dm * dm, axis=1, keepdims=True)
        misc = misc_ref[...]
        m1 = (s_ref[...] + rad * misc[0:1, :]
              + _dot(ef_ref[...], w1d_ref[...], ((1,), (0,)))
              + misc[1:2, :])
        m1 = _silu(m1)
        m2 = _silu(_dot(m1, w2_ref[...], ((1,), (0,))) + misc[2:3, :])
        att = jax.nn.sigmoid(
            jnp.sum(m2 * misc[3:4, :], axis=1, keepdims=True) + misc[4, 0])
        mv = m2 * att
        cm = _silu(_dot(mv, wc1_ref[...], ((1,), (0,))) + misc[5:6, :])
        ct = jnp.tanh(jnp.sum(cm * misc[6:7, :], axis=1, keepdims=True))
        m_ref[...] = mv
        tr_ref[...] = jnp.where(lane == 3, 1.0, dm * ct)

    return pl.pallas_call(
        body,
        grid=(e // blk,),
        in_specs=[pl.BlockSpec((blk, 128), lambda i: (i, 0)),
                  pl.BlockSpec((blk, 16), lambda i: (i, 0)),
                  pl.BlockSpec((blk, 32), lambda i: (i, 0)),
                  pl.BlockSpec((32, 128), lambda i: (0, 0)),
                  pl.BlockSpec((128, 128), lambda i: (0, 0)),
                  pl.BlockSpec((128, 128), lambda i: (0, 0)),
                  pl.BlockSpec((8, 128), lambda i: (0, 0))],
        out_specs=[pl.BlockSpec((blk, 128), lambda i: (i, 0)),
                   pl.BlockSpec((blk, 16), lambda i: (i, 0))],
        out_shape=[jax.ShapeDtypeStruct((e, 128), jnp.float32),
                   jax.ShapeDtypeStruct((e, 16), jnp.float32)],
    )(s, diff, ef, w1d, w2, wc1, misc)


def _node_mlp(h, p, wa, wb, w2, bb, blk=2000):
    n = h.shape[0]

    def body(h_ref, p_ref, wa_ref, wb_ref, w2_ref, b_ref, o_ref):
        hv = h_ref[...]
        magg = p_ref[0] + p_ref[1]
        x = (_dot(hv, wa_ref[...], ((1,), (0,)))
             + _dot(magg, wb_ref[...], ((1,), (0,))) + b_ref[0:1, :])
        x = _silu(x)
        o_ref[...] = _dot(x, w2_ref[...], ((1,), (0,))) + b_ref[1:2, :] + hv

    return pl.pallas_call(
        body,
        grid=(n // blk,),
        in_specs=[pl.BlockSpec((blk, 128), lambda i: (i, 0)),
                  pl.BlockSpec((2, blk, 128), lambda i: (0, i, 0)),
                  pl.BlockSpec((128, 128), lambda i: (0, 0)),
                  pl.BlockSpec((128, 128), lambda i: (0, 0)),
                  pl.BlockSpec((128, 128), lambda i: (0, 0)),
                  pl.BlockSpec((2, 128), lambda i: (0, 0))],
        out_specs=pl.BlockSpec((blk, 128), lambda i: (i, 0)),
        out_shape=jax.ShapeDtypeStruct((n, 128), jnp.float32),
    )(h, p, wa, wb, w2, bb)


def _coord_update(c, t, blk=200):
    n = c.shape[0]

    def body(c_ref, t_ref, o_ref):
        tv = t_ref[0] + t_ref[1]
        cnt = jnp.maximum(tv[:, 3:4], 1.0)
        o_ref[...] = c_ref[...] + tv[:, :3] / cnt

    return pl.pallas_call(
        body,
        grid=(n // blk,),
        in_specs=[pl.BlockSpec((blk, 3), lambda i: (i, 0)),
                  pl.BlockSpec((2, blk, 16), lambda i: (0, i, 0))],
        out_specs=pl.BlockSpec((blk, 3), lambda i: (i, 0)),
        out_shape=jax.ShapeDtypeStruct((n, 3), jnp.float32),
    )(c, t)


def _gcl(lp, h, c, row, col, ef, N, E):
    w1 = lp["edge_mlp"][0]["W"]
    misc = jnp.zeros((8, 128), jnp.float32)
    misc = misc.at[0].set(w1[256])
    misc = misc.at[1].set(lp["edge_mlp"][0]["b"])
    misc = misc.at[2].set(lp["edge_mlp"][1]["b"])
    misc = misc.at[3].set(lp["att_mlp"]["W"][:, 0])
    misc = misc.at[4, 0].set(lp["att_mlp"]["b"][0])
    misc = misc.at[5].set(lp["coord_mlp"][0]["b"])
    misc = misc.at[6].set(lp["coord_mlp"][1]["W"][:, 0])

    a, b = _ab_proj(h, w1[:128], w1[128:256])
    cflat = jnp.pad(c.reshape(3 * N), (0, _tl3(N) - 3 * N))
    s, diff = _make_gather(N, E)(a, b, cflat, row, col)
    m, tr = _edge_mlp(s, diff, ef, w1[257:289],
                      lp["edge_mlp"][1]["W"], lp["coord_mlp"][0]["W"], misc)
    p, t = _make_scatter(N, E)(m, tr.reshape(16 * E), row)
    bb = jnp.stack([lp["node_mlp"][0]["b"], lp["node_mlp"][1]["b"]])
    wn1 = lp["node_mlp"][0]["W"]
    h = _node_mlp(h, p, wn1[:128], wn1[128:256], lp["node_mlp"][1]["W"], bb)
    c = _coord_update(c, t.reshape(2, -1, 16))
    return h, c


def _egnn(p, h, c, row, col, ef, N, E):
    h = _linear_pallas(h, p["emb_in"]["W"], p["emb_in"]["b"], "none", 2000)
    for lp in p["layers"]:
        h, c = _gcl(lp, h, c, row, col, ef, N, E)
    return _linear_pallas(h, p["emb_out"]["W"], p["emb_out"]["b"],
                          "none", 2000), c


def kernel(x_res, x_pos, edge_feat, edge_index, params):
    N = x_res.shape[0]
    E = edge_feat.shape[0]
    row = edge_index[0]
    col = edge_index[1]
    c = x_pos.astype(jnp.float32)
    ef = _linear_pallas(edge_feat, params["edge_fc"]["W"],
                        params["edge_fc"]["b"], "none", 3200)
    h1, c = _egnn(params["eg1"], x_res, c, row, col, ef, N, E)
    h2, c = _egnn(params["eg2"], h1, c, row, col, ef, N, E)
    h3, c = _egnn(params["eg3"], h2, c, row, col, ef, N, E)
    h4, c = _egnn(params["eg4"], h3, c, row, col, ef, N, E)
    out1 = _fc_head(h1, params["fc1"])
    out2 = _fc_head(h2, params["fc2"])
    out3 = _fc_head(h3, params["fc3"])
    out4 = _fc_head(h4, params["fc4"])
    return (out4, out3, out2, out1, h4, h3, h2, h1)
